# baseline (speedup 1.0000x reference)
import functools

import jax
import jax.numpy as jnp
from jax import lax
from jax.experimental import pallas as pl
from jax.experimental.pallas import tpu as pltpu
from jax.experimental.pallas import tpu_sc as plsc

NUM_BUCKETS = 100000
MULT = 1000003
EMBED_DIM = 128
MODEL_DIM = 2048
BATCH = 4
SEQ = 4096
TOKENS = BATCH * SEQ

NC, NS, L = 2, 16, 16
NW = NC * NS

CHUNKS = ((0, 4096), (4096, 4096), (8192, 8192))

BM = 1024


def _gather_rows(tpw):
    for g in range(128, 0, -16):
        if tpw % g == 0:
            return g
    raise ValueError(tpw)


def _sc_hash_gather_chunk(ids, emb, start, ct):
    tpw = ct // NW
    gch = _gather_rows(tpw)
    ng = tpw // gch
    mesh = plsc.VectorSubcoreMesh(
        core_axis_name="c", subcore_axis_name="s", num_cores=NC, num_subcores=NS
    )

    @functools.partial(
        pl.kernel,
        out_type=jax.ShapeDtypeStruct((ct, EMBED_DIM), jnp.float32),
        mesh=mesh,
        scratch_types=[
            pltpu.VMEM((8 + tpw,), jnp.int32),
            [pltpu.VMEM((gch,), jnp.int32) for _ in range(ng)],
            pltpu.VMEM((tpw, EMBED_DIM), jnp.float32),
            [pltpu.SemaphoreType.DMA for _ in range(ng)],
            pltpu.SemaphoreType.DMA,
        ],
        name=f"hash_gather_t{start}",
    )
    def hash_gather(ids_hbm, emb_hbm, out_hbm, win_v, idx_vs, rows_v, gsems, wsem):
        wid = lax.axis_index("s") * NC + lax.axis_index("c")
        base = start + wid * tpw
        d1 = pltpu.async_copy(
            ids_hbm.at[pl.ds(base, tpw)], win_v.at[pl.ds(8, tpw)], gsems[0]
        )
        head = pl.multiple_of(jnp.maximum(base - 8, 0), 8)
        d2 = pltpu.async_copy(
            ids_hbm.at[pl.ds(head, 8)], win_v.at[pl.ds(0, 8)], wsem
        )
        d1.wait()
        d2.wait()
        gcopies = []
        for g in range(ng):
            for jj in range(gch // L):
                j = g * (gch // L) + jj
                cur = win_v[pl.ds(8 + j * L, L)]
                prv = win_v[pl.ds(7 + j * L, L)]
                pos = base + j * L + lax.iota(jnp.int32, L)
                prv = jnp.where(lax.rem(pos, SEQ) == 0, 0, prv)
                h = prv * MULT + cur
                r = lax.rem(h, NUM_BUCKETS)
                r = jnp.where(r < 0, r + NUM_BUCKETS, r)
                idx_vs[g][pl.ds(jj * L, L)] = r
            gcopies.append(
                pltpu.async_copy(
                    emb_hbm.at[idx_vs[g]], rows_v.at[pl.ds(g * gch, gch)], gsems[g]
                )
            )
        wcopies = []
        for g in range(ng):
            gcopies[g].wait()
            wcopies.append(
                pltpu.async_copy(
                    rows_v.at[pl.ds(g * gch, gch)],
                    out_hbm.at[pl.ds(wid * tpw + g * gch, gch)],
                    wsem,
                )
            )
        for w in wcopies:
            w.wait()

    return hash_gather(ids, emb)


def _tc_matmul_chunk(x_c, w, out_prev, start, ct):
    def body(x_ref, w_ref, *rest):
        o_ref = rest[-1]
        o_ref[...] = lax.dot_general(
            x_ref[...], w_ref[...],
            (((1,), (1,)), ((), ())),
            preferred_element_type=jnp.float32,
        )

    in_specs = [
        pl.BlockSpec((BM, EMBED_DIM), lambda i: (i, 0)),
        pl.BlockSpec((MODEL_DIM, EMBED_DIM), lambda i: (0, 0)),
    ]
    args = [x_c, w]
    kwargs = {}
    if out_prev is not None:
        in_specs.append(pl.BlockSpec(memory_space=pl.ANY))
        args.append(out_prev)
        kwargs = dict(input_output_aliases={2: 0})
    row0 = start // BM
    return pl.pallas_call(
        body,
        grid=(ct // BM,),
        in_specs=in_specs,
        out_specs=pl.BlockSpec((BM, MODEL_DIM), lambda i: (row0 + i, 0)),
        out_shape=jax.ShapeDtypeStruct((TOKENS, MODEL_DIM), jnp.float32),
        **kwargs,
    )(*args)


def kernel(input_ids, emb, W):
    ids = input_ids.astype(jnp.int32).reshape(-1)
    xs = [_sc_hash_gather_chunk(ids, emb, s, n) for s, n in CHUNKS]
    out = None
    for x_c, (s, n) in zip(xs, CHUNKS):
        out = _tc_matmul_chunk(x_c, W, out, s, n)
    return out.reshape(BATCH, SEQ, MODEL_DIM)


# speedup vs baseline: 1.0479x; 1.0479x over previous
import functools

import jax
import jax.numpy as jnp
from jax import lax
from jax.experimental import pallas as pl
from jax.experimental.pallas import tpu as pltpu
from jax.experimental.pallas import tpu_sc as plsc

NUM_BUCKETS = 100000
MULT = 1000003
EMBED_DIM = 128
MODEL_DIM = 2048
BATCH = 4
SEQ = 4096
TOKENS = BATCH * SEQ

NC, NS, L = 2, 16, 16
NW = NC * NS

CHUNKS = ((0, 5120), (5120, 11264))

BM = 1024


def _gather_rows(tpw):
    for g in range(128, 0, -16):
        if tpw % g == 0:
            return g
    raise ValueError(tpw)


def _sc_hash_gather_chunk(ids, emb, start, ct):
    tpw = ct // NW
    gch = _gather_rows(tpw)
    ng = tpw // gch
    mesh = plsc.VectorSubcoreMesh(
        core_axis_name="c", subcore_axis_name="s", num_cores=NC, num_subcores=NS
    )

    @functools.partial(
        pl.kernel,
        out_type=jax.ShapeDtypeStruct((ct, EMBED_DIM), jnp.float32),
        mesh=mesh,
        scratch_types=[
            pltpu.VMEM((8 + tpw,), jnp.int32),
            [pltpu.VMEM((gch,), jnp.int32) for _ in range(ng)],
            pltpu.VMEM((tpw, EMBED_DIM), jnp.float32),
            [pltpu.SemaphoreType.DMA for _ in range(ng)],
            pltpu.SemaphoreType.DMA,
        ],
        name=f"hash_gather_t{start}",
    )
    def hash_gather(ids_hbm, emb_hbm, out_hbm, win_v, idx_vs, rows_v, gsems, wsem):
        wid = lax.axis_index("s") * NC + lax.axis_index("c")
        base = start + wid * tpw
        d1 = pltpu.async_copy(
            ids_hbm.at[pl.ds(base, tpw)], win_v.at[pl.ds(8, tpw)], gsems[0]
        )
        head = pl.multiple_of(jnp.maximum(base - 8, 0), 8)
        d2 = pltpu.async_copy(
            ids_hbm.at[pl.ds(head, 8)], win_v.at[pl.ds(0, 8)], wsem
        )
        d1.wait()
        d2.wait()
        gcopies = []
        for g in range(ng):
            for jj in range(gch // L):
                j = g * (gch // L) + jj
                cur = win_v[pl.ds(8 + j * L, L)]
                prv = win_v[pl.ds(7 + j * L, L)]
                pos = base + j * L + lax.iota(jnp.int32, L)
                prv = jnp.where(lax.rem(pos, SEQ) == 0, 0, prv)
                h = prv * MULT + cur
                r = lax.rem(h, NUM_BUCKETS)
                r = jnp.where(r < 0, r + NUM_BUCKETS, r)
                idx_vs[g][pl.ds(jj * L, L)] = r
            gcopies.append(
                pltpu.async_copy(
                    emb_hbm.at[idx_vs[g]], rows_v.at[pl.ds(g * gch, gch)], gsems[g]
                )
            )
        wcopies = []
        for g in range(ng):
            gcopies[g].wait()
            wcopies.append(
                pltpu.async_copy(
                    rows_v.at[pl.ds(g * gch, gch)],
                    out_hbm.at[pl.ds(wid * tpw + g * gch, gch)],
                    wsem,
                )
            )
        for w in wcopies:
            w.wait()

    return hash_gather(ids, emb)


def _tc_matmul_chunk(x_c, w, out_prev, start, ct):
    def body(x_ref, w_ref, *rest):
        o_ref = rest[-1]
        o_ref[...] = lax.dot_general(
            x_ref[...], w_ref[...],
            (((1,), (1,)), ((), ())),
            preferred_element_type=jnp.float32,
        )

    in_specs = [
        pl.BlockSpec((BM, EMBED_DIM), lambda i: (i, 0)),
        pl.BlockSpec((MODEL_DIM, EMBED_DIM), lambda i: (0, 0)),
    ]
    args = [x_c, w]
    kwargs = {}
    if out_prev is not None:
        in_specs.append(pl.BlockSpec(memory_space=pl.ANY))
        args.append(out_prev)
        kwargs = dict(input_output_aliases={2: 0})
    row0 = start // BM
    return pl.pallas_call(
        body,
        grid=(ct // BM,),
        in_specs=in_specs,
        out_specs=pl.BlockSpec((BM, MODEL_DIM), lambda i: (row0 + i, 0)),
        out_shape=jax.ShapeDtypeStruct((TOKENS, MODEL_DIM), jnp.float32),
        **kwargs,
    )(*args)


def kernel(input_ids, emb, W):
    ids = input_ids.astype(jnp.int32).reshape(-1)
    xs = [_sc_hash_gather_chunk(ids, emb, s, n) for s, n in CHUNKS]
    out = None
    for x_c, (s, n) in zip(xs, CHUNKS):
        out = _tc_matmul_chunk(x_c, W, out, s, n)
    return out.reshape(BATCH, SEQ, MODEL_DIM)
